# Initial kernel scaffold; baseline (speedup 1.0000x reference)
import jax
import jax.numpy as jnp
from jax.experimental import pallas as pl


def kernel(c0, c2, l0, l1, l2, sd, inds_back):
    raise NotImplementedError("write your pallas kernel here")


# speedup vs baseline: 1.0885x; 1.0885x over previous
import functools

import jax
import jax.numpy as jnp
from jax import lax
from jax.experimental import pallas as pl
from jax.experimental.pallas import tpu as pltpu
from jax.experimental.pallas import tpu_sc as plsc

N_VERTS = 3889
N_CENTER = 889
N_LEFT = 1500
N_SD = 20
N_FIXED = 10
SEC = 32
NSEC = 4
PAD_B = 4096

_info = plsc.get_sparse_core_info()
_NC = _info.num_cores
_NS = _info.num_subcores
_NW = _NC * _NS
_BPW = PAD_B // _NW


def _build_table_body(c0_ref, c2_ref, l0_ref, l1_ref, l2_ref, out_ref):
    out_ref[...] = jnp.zeros((N_VERTS, NSEC * SEC), jnp.float32)
    a, b = N_CENTER, N_CENTER + N_LEFT
    out_ref[0:a, 0:N_SD] = c0_ref[...]
    out_ref[0:a, 2 * SEC:2 * SEC + N_SD] = c2_ref[...]
    out_ref[a:b, 0:N_SD] = l0_ref[...]
    out_ref[a:b, SEC:SEC + N_SD] = l1_ref[...]
    out_ref[a:b, 2 * SEC:2 * SEC + N_SD] = l2_ref[...]
    out_ref[b:N_VERTS, 0:N_SD] = l0_ref[...]
    out_ref[b:N_VERTS, SEC:SEC + N_SD] = -l1_ref[...]
    out_ref[b:N_VERTS, 2 * SEC:2 * SEC + N_SD] = l2_ref[...]


_sc_mesh = plsc.VectorSubcoreMesh(core_axis_name="c", subcore_axis_name="s")


@functools.partial(
    pl.kernel,
    mesh=_sc_mesh,
    out_type=jax.ShapeDtypeStruct((PAD_B, NSEC * SEC), jnp.float32),
    scratch_types=[
        pltpu.VMEM((_BPW,), jnp.int32),
        pltpu.VMEM((_BPW, NSEC * SEC), jnp.float32),
        pltpu.SemaphoreType.DMA,
    ],
)
def _sc_gather(table_hbm, idx_hbm, out_hbm, idx_v, rows_v, sem):
    wid = lax.axis_index("s") * _NC + lax.axis_index("c")
    base = wid * _BPW
    pltpu.sync_copy(idx_hbm.at[pl.ds(base, _BPW)], idx_v)
    pltpu.async_copy(table_hbm.at[idx_v], rows_v, sem).wait()
    pltpu.sync_copy(rows_v, out_hbm.at[pl.ds(base, _BPW)])


def _assemble_body(sd_ref, g_ref, comp_ref, prep_ref):
    sdh = sd_ref[:, :, 0:N_FIXED]
    gg = g_ref[0:N_VERTS, 0:3, 0:N_SD]
    comp = jnp.concatenate([sdh, gg], axis=2)
    comp_ref[...] = comp
    flat = comp.reshape(N_VERTS * 3, 30)
    rows = lax.broadcasted_iota(jnp.int32, (30, 30), 0)
    cols = lax.broadcasted_iota(jnp.int32, (30, 30), 1)
    eye = (rows == cols).astype(jnp.float32)
    prep_ref[...] = lax.dot_general(
        eye, flat, (((1,), (1,)), ((), ())),
        preferred_element_type=jnp.float32,
        precision=lax.Precision.HIGHEST,
    )


def kernel(c0, c2, l0, l1, l2, sd, inds_back):
    table = pl.pallas_call(
        _build_table_body,
        out_shape=jax.ShapeDtypeStruct((N_VERTS, NSEC * SEC), jnp.float32),
    )(c0, c2, l0, l1, l2)

    idx = jnp.pad(inds_back.astype(jnp.int32), (0, PAD_B - N_VERTS))
    g = _sc_gather(table, idx).reshape(PAD_B, NSEC, SEC)

    comp, prep = pl.pallas_call(
        _assemble_body,
        out_shape=(
            jax.ShapeDtypeStruct((N_VERTS, 3, 30), jnp.float32),
            jax.ShapeDtypeStruct((30, N_VERTS * 3), jnp.float32),
        ),
    )(sd, g)
    return comp, prep


# speedup vs baseline: 1.1036x; 1.0138x over previous
import functools

import jax
import jax.numpy as jnp
from jax import lax
from jax.experimental import pallas as pl
from jax.experimental.pallas import tpu as pltpu
from jax.experimental.pallas import tpu_sc as plsc

N_VERTS = 3889
N_CENTER = 889
N_LEFT = 1500
N_SD = 20
N_FIXED = 10
SEC = 32
ROW = 128
PAD_B = 4096

_info = plsc.get_sparse_core_info()
_NC = _info.num_cores
_NS = _info.num_subcores
_NW = _NC * _NS
_BPW = PAD_B // _NW


def _build_table_body(c0_ref, c2_ref, l0_ref, l1_ref, l2_ref, idx_ref,
                      out_ref, idxp_ref):
    out_ref[...] = jnp.zeros((N_VERTS, ROW), jnp.float32)
    a, b = N_CENTER, N_CENTER + N_LEFT
    out_ref[0:a, 0:N_SD] = c0_ref[...]
    out_ref[0:a, 2 * SEC:2 * SEC + N_SD] = c2_ref[...]
    out_ref[a:b, 0:N_SD] = l0_ref[...]
    out_ref[a:b, SEC:SEC + N_SD] = l1_ref[...]
    out_ref[a:b, 2 * SEC:2 * SEC + N_SD] = l2_ref[...]
    out_ref[b:N_VERTS, 0:N_SD] = l0_ref[...]
    out_ref[b:N_VERTS, SEC:SEC + N_SD] = -l1_ref[...]
    out_ref[b:N_VERTS, 2 * SEC:2 * SEC + N_SD] = l2_ref[...]
    idxp_ref[...] = jnp.zeros((1, PAD_B), jnp.int32)
    idxp_ref[0, 0:N_VERTS] = idx_ref[0, :]


_sc_mesh = plsc.VectorSubcoreMesh(core_axis_name="c", subcore_axis_name="s")


@functools.partial(
    pl.kernel,
    mesh=_sc_mesh,
    out_type=jax.ShapeDtypeStruct((PAD_B, ROW), jnp.float32),
    scratch_types=[
        pltpu.VMEM((_BPW,), jnp.int32),
        pltpu.VMEM((_BPW, ROW), jnp.float32),
        pltpu.SemaphoreType.DMA,
    ],
)
def _sc_gather(table_hbm, idx_hbm, out_hbm, idx_v, rows_v, sem):
    wid = lax.axis_index("s") * _NC + lax.axis_index("c")
    base = wid * _BPW
    pltpu.sync_copy(idx_hbm.at[pl.ds(base, _BPW)], idx_v)
    pltpu.async_copy(table_hbm.at[idx_v], rows_v, sem).wait()
    pltpu.sync_copy(rows_v, out_hbm.at[pl.ds(base, _BPW)])


def _assemble_body(sd_ref, g_ref, comp_ref, prep_ref):
    sdh = sd_ref[:, :, 0:N_FIXED]
    gg = g_ref[0:N_VERTS, 0:3, 0:N_SD]
    comp = jnp.concatenate([sdh, gg], axis=2)
    comp_ref[...] = comp
    flat = comp.reshape(N_VERTS * 3, 30)
    r = lax.broadcasted_iota(jnp.int32, (30, 30), 0)
    c = lax.broadcasted_iota(jnp.int32, (30, 30), 1)
    eye = (r == c).astype(jnp.float32)
    prep_ref[...] = lax.dot_general(
        eye, flat, (((1,), (1,)), ((), ())),
        preferred_element_type=jnp.float32,
    )


def kernel(c0, c2, l0, l1, l2, sd, inds_back):
    idx2d = inds_back.astype(jnp.int32).reshape(1, N_VERTS)
    table, idxp = pl.pallas_call(
        _build_table_body,
        out_shape=(
            jax.ShapeDtypeStruct((N_VERTS, ROW), jnp.float32),
            jax.ShapeDtypeStruct((1, PAD_B), jnp.int32),
        ),
    )(c0, c2, l0, l1, l2, idx2d)

    g = _sc_gather(table, idxp.reshape(PAD_B)).reshape(PAD_B, ROW // SEC, SEC)

    comp, prep = pl.pallas_call(
        _assemble_body,
        out_shape=(
            jax.ShapeDtypeStruct((N_VERTS, 3, 30), jnp.float32),
            jax.ShapeDtypeStruct((30, N_VERTS * 3), jnp.float32),
        ),
    )(sd, g)
    return comp, prep
